# Initial kernel scaffold; baseline (speedup 1.0000x reference)
import jax
import jax.numpy as jnp
from jax.experimental import pallas as pl


def kernel(final_mat, embeddings_table):
    raise NotImplementedError("write your pallas kernel here")


# speedup vs baseline: 1.3721x; 1.3721x over previous
import functools

import jax
import jax.numpy as jnp
from jax import lax
from jax.experimental import pallas as pl
from jax.experimental.pallas import tpu as pltpu
from jax.experimental.pallas import tpu_sc as plsc

NUM_UNITS = 64
TABLE_ROWS = 257
SEQ = 2048
B = SEQ * SEQ
NC, NS, L = 2, 16, 16
NW = NC * NS
BLOCK = 512
PER_W = B // NW
N_BLOCKS = PER_W // BLOCK
GROUPS = BLOCK // L


def _make_sc_gather():
    mesh = plsc.VectorSubcoreMesh(core_axis_name="c", subcore_axis_name="s")

    @functools.partial(
        pl.kernel,
        mesh=mesh,
        compiler_params=pltpu.CompilerParams(needs_layout_passes=False),
        out_type=jax.ShapeDtypeStruct((B * NUM_UNITS,), jnp.float32),
        scratch_types=[
            pltpu.VMEM((TABLE_ROWS * NUM_UNITS,), jnp.float32),
            pltpu.VMEM((BLOCK,), jnp.int32),
            pltpu.VMEM((BLOCK * NUM_UNITS,), jnp.float32),
        ],
    )
    def sc_gather(fm_hbm, table_hbm, out_hbm, table_v, idx_v, rows_v):
        wid = lax.axis_index("s") * NC + lax.axis_index("c")
        base = wid * PER_W
        pltpu.sync_copy(table_hbm, table_v)
        lane64 = lax.iota(jnp.int32, L) * NUM_UNITS

        def block_body(blk, carry):
            off = base + blk * BLOCK
            pltpu.sync_copy(fm_hbm.at[pl.ds(off, BLOCK)], idx_v)

            def group_body(g, c):
                iv = idx_v[pl.ds(g * L, L)]
                ab = iv * NUM_UNITS
                sb = lane64 + g * (L * NUM_UNITS)
                for d in range(NUM_UNITS):
                    v = plsc.load_gather(table_v, [ab + d])
                    plsc.store_scatter(rows_v, [sb + d], v)
                return c

            lax.fori_loop(0, GROUPS, group_body, 0)
            pltpu.sync_copy(
                rows_v, out_hbm.at[pl.ds(off * NUM_UNITS, BLOCK * NUM_UNITS)]
            )
            return carry

        lax.fori_loop(0, N_BLOCKS, block_body, 0)

    return sc_gather


_sc_gather = _make_sc_gather()


def kernel(final_mat, embeddings_table):
    fm = final_mat.reshape(B).astype(jnp.int32)
    out = _sc_gather(fm, embeddings_table.reshape(-1))
    return out.reshape(SEQ, SEQ, NUM_UNITS)


# speedup vs baseline: 5.1786x; 3.7741x over previous
import functools

import jax
import jax.numpy as jnp
from jax import lax
from jax.experimental import pallas as pl
from jax.experimental.pallas import tpu as pltpu
from jax.experimental.pallas import tpu_sc as plsc

NUM_UNITS = 64
TABLE_ROWS = 257
SEQ = 2048
B = SEQ * SEQ
NC, NS, L = 2, 16, 16
NW = NC * NS
BLOCK = 512
BLK_PER_ROW = SEQ // BLOCK
PER_W = B // NW
ROWS_PER_W = SEQ // NW
N_BLOCKS = PER_W // BLOCK
GROUPS = BLOCK // L


def _make_sc_gather():
    mesh = plsc.VectorSubcoreMesh(core_axis_name="c", subcore_axis_name="s")

    @functools.partial(
        pl.kernel,
        mesh=mesh,
        compiler_params=pltpu.CompilerParams(
            needs_layout_passes=False, use_tc_tiling_on_sc=False
        ),
        out_type=jax.ShapeDtypeStruct((SEQ, SEQ, NUM_UNITS), jnp.float32),
        scratch_types=[
            pltpu.VMEM((TABLE_ROWS * NUM_UNITS,), jnp.float32),
            pltpu.VMEM((BLOCK,), jnp.int32),
            pltpu.VMEM((BLOCK,), jnp.int32),
            pltpu.VMEM((BLOCK, NUM_UNITS), jnp.float32),
            pltpu.VMEM((BLOCK, NUM_UNITS), jnp.float32),
            pltpu.SemaphoreType.DMA,
            pltpu.SemaphoreType.DMA,
        ],
    )
    def sc_gather(fm_hbm, table_hbm, out_hbm, table_v, idx0, idx1, rows0,
                  rows1, sem0, sem1):
        wid = lax.axis_index("s") * NC + lax.axis_index("c")
        base = wid * PER_W
        row_base = wid * ROWS_PER_W
        pltpu.sync_copy(table_hbm, table_v)

        def expand(idx_v, rows_v, blk):
            off = base + blk * BLOCK
            pltpu.sync_copy(fm_hbm.at[pl.ds(off, BLOCK)], idx_v)

            def group_body(g, c):
                iv = idx_v[pl.ds(g * L, L)] * NUM_UNITS
                for r in range(L):
                    src = iv[r]
                    b = g * L + r
                    vals = [
                        table_v[pl.ds(src + k, L)]
                        for k in range(0, NUM_UNITS, L)
                    ]
                    for k, v in zip(range(0, NUM_UNITS, L), vals):
                        rows_v[b, pl.ds(k, L)] = v
                return c

            lax.fori_loop(0, GROUPS, group_body, 0)

        def out_slice(blk):
            i = row_base + blk // BLK_PER_ROW
            j0 = (blk % BLK_PER_ROW) * BLOCK
            return out_hbm.at[i, pl.ds(j0, BLOCK)]

        def pair_body(i, carry):
            blk0 = 2 * i
            blk1 = blk0 + 1

            @pl.when(i > 0)
            def _():
                pltpu.make_async_copy(rows0, out_slice(blk0), sem0).wait()

            expand(idx0, rows0, blk0)
            pltpu.async_copy(rows0, out_slice(blk0), sem0)

            @pl.when(i > 0)
            def _():
                pltpu.make_async_copy(rows1, out_slice(blk1), sem1).wait()

            expand(idx1, rows1, blk1)
            pltpu.async_copy(rows1, out_slice(blk1), sem1)
            return carry

        lax.fori_loop(0, N_BLOCKS // 2, pair_body, 0)
        pltpu.make_async_copy(rows0, out_slice(0), sem0).wait()
        pltpu.make_async_copy(rows1, out_slice(1), sem1).wait()

    return sc_gather


_sc_gather = _make_sc_gather()


def kernel(final_mat, embeddings_table):
    fm = final_mat.reshape(B).astype(jnp.int32)
    return _sc_gather(fm, embeddings_table.reshape(-1))
